# speedup vs baseline: 1.3129x; 1.3129x over previous
import functools

import jax
import jax.numpy as jnp
from jax import lax
from jax.experimental import pallas as pl
from jax.experimental.pallas import tpu as pltpu
from jax.experimental.pallas import tpu_sc as plsc

BATCH = 128
SEQ = 200
D = 2048
TOTAL = BATCH * SEQ
NC = 2
NS = 16
NW = NC * NS
B_PER_W = TOTAL // NW
CHUNK = 16
NCHUNKS = B_PER_W // CHUNK


def _body(idx_hbm, table_hbm, out_hbm, idx_v, rows_v, gsem):
    wid = lax.axis_index("s") * NC + lax.axis_index("c")
    base = wid * B_PER_W
    pltpu.sync_copy(idx_hbm.at[wid], idx_v)

    def chunk(j, carry):
        pltpu.async_copy(table_hbm.at[idx_v.at[j]], rows_v, gsem).wait()
        pltpu.sync_copy(rows_v, out_hbm.at[pl.ds(base + j * CHUNK, CHUNK)])
        return carry

    lax.fori_loop(0, NCHUNKS, chunk, 0)


_gather = functools.partial(
    pl.kernel,
    mesh=plsc.VectorSubcoreMesh(core_axis_name="c", subcore_axis_name="s"),
    out_type=jax.ShapeDtypeStruct((TOTAL, D), jnp.float32),
    scratch_types=[
        pltpu.VMEM((NCHUNKS, CHUNK), jnp.int32),
        pltpu.VMEM((CHUNK, D), jnp.float32),
        pltpu.SemaphoreType.DMA,
    ],
)(_body)


def kernel(indices, weight):
    idx = indices.astype(jnp.int32).reshape(NW, NCHUNKS, CHUNK)
    out = _gather(idx, weight)
    return out.reshape(BATCH, SEQ, D)


# speedup vs baseline: 1.4141x; 1.0771x over previous
import functools

import jax
import jax.numpy as jnp
from jax import lax
from jax.experimental import pallas as pl
from jax.experimental.pallas import tpu as pltpu
from jax.experimental.pallas import tpu_sc as plsc

BATCH = 128
SEQ = 200
D = 2048
TOTAL = BATCH * SEQ
NC = 2
NS = 16
NW = NC * NS
B_PER_W = TOTAL // NW
CHUNK = 16
NCHUNKS = B_PER_W // CHUNK


NPAIRS = NCHUNKS // 2


def _body(idx_hbm, table_hbm, out_hbm, idx_v, rows_v, gsem, wsem):
    wid = lax.axis_index("s") * NC + lax.axis_index("c")
    base = wid * B_PER_W
    pltpu.sync_copy(idx_hbm.at[wid], idx_v)

    def g_copy(j, b):
        return pltpu.make_async_copy(table_hbm.at[idx_v.at[j]], rows_v.at[b], gsem)

    def w_copy(j, b):
        return pltpu.make_async_copy(
            rows_v.at[b], out_hbm.at[pl.ds(base + j * CHUNK, CHUNK)], wsem)

    g_copy(0, 0).start()

    def pair(p, carry):
        j0 = 2 * p
        g_copy(j0, 0).wait()
        g_copy(j0 + 1, 1).start()
        w_copy(j0, 0).start()
        g_copy(j0 + 1, 1).wait()
        w_copy(j0, 0).wait()

        @pl.when(p + 1 < NPAIRS)
        def _():
            g_copy(j0 + 2, 0).start()

        w_copy(j0 + 1, 1).start()
        w_copy(j0 + 1, 1).wait()
        return carry

    lax.fori_loop(0, NPAIRS, pair, 0)


_gather = functools.partial(
    pl.kernel,
    mesh=plsc.VectorSubcoreMesh(core_axis_name="c", subcore_axis_name="s"),
    out_type=jax.ShapeDtypeStruct((TOTAL, D), jnp.float32),
    scratch_types=[
        pltpu.VMEM((NCHUNKS, CHUNK), jnp.int32),
        pltpu.VMEM((2, CHUNK, D), jnp.float32),
        pltpu.SemaphoreType.DMA,
        pltpu.SemaphoreType.DMA,
    ],
)(_body)


def kernel(indices, weight):
    idx = indices.astype(jnp.int32).reshape(NW, NCHUNKS, CHUNK)
    out = _gather(idx, weight)
    return out.reshape(BATCH, SEQ, D)
